# baseline (speedup 1.0000x reference)
import functools

import jax
import jax.numpy as jnp
from jax import lax
from jax.experimental import pallas as pl
from jax.experimental.pallas import tpu as pltpu
from jax.experimental.pallas import tpu_sc as plsc

FIELD_NUM = 26
LATENT_DIM = 16
EPSILON = 1e-3

NUM_CORES = 2
NUM_SUBCORES = 16
NUM_WORKERS = NUM_CORES * NUM_SUBCORES

BLK = 128
CH_BLOCKS = 4
CH = BLK * CH_BLOCKS


@functools.lru_cache(maxsize=None)
def _build(batch):
    n_rows = batch * FIELD_NUM
    per_w = n_rows // NUM_WORKERS
    n_chunks = per_w // CH
    assert per_w % CH == 0 and batch % BLK == 0
    out_rows = n_rows * LATENT_DIM // 128
    bt_per_f = batch // BLK
    assert batch & (batch - 1) == 0
    bshift = batch.bit_length() - 1
    mesh = plsc.VectorSubcoreMesh(core_axis_name="c", subcore_axis_name="s")

    @functools.partial(
        pl.kernel,
        mesh=mesh,
        out_type=jax.ShapeDtypeStruct((out_rows, 128), jnp.float32),
        compiler_params=pltpu.CompilerParams(
            use_tc_tiling_on_sc=False, needs_layout_passes=False
        ),
        scratch_types=[
            pltpu.VMEM((CH,), jnp.int32),
            pltpu.VMEM((CH, LATENT_DIM), jnp.float32),
            pltpu.VMEM((2, CH_BLOCKS * 8, 128), jnp.float32),
            pltpu.VMEM((FIELD_NUM * LATENT_DIM,), jnp.float32),
            pltpu.SemaphoreType.DMA,
        ],
    )
    def k(idxf_hbm, arch_hbm, table_hbm, out_hbm, idx_v, rows_v, obuf, arch_v, sem):
        wid = lax.axis_index("s") * NUM_CORES + lax.axis_index("c")
        p0w = wid * per_w
        pltpu.sync_copy(arch_hbm, arch_v)
        iota = lax.iota(jnp.int32, 16)

        def chunk_body(c, carry):
            p0 = p0w + c * CH
            f = lax.shift_right_logical(p0, bshift)
            bt0 = lax.shift_right_logical(p0 & (batch - 1), 7)
            pltpu.sync_copy(idxf_hbm.at[pl.ds(p0, CH)], idx_v)
            cps = [
                pltpu.async_copy(
                    table_hbm.at[idx_v.at[pl.ds(j * BLK, BLK)]],
                    rows_v.at[pl.ds(j * BLK, BLK)],
                    sem,
                )
                for j in range(CH_BLOCKS)
            ]
            for cp in cps:
                cp.wait()
            a = arch_v[pl.ds(f * LATENT_DIM, LATENT_DIM)]
            a2 = a * a
            g = a2 / (a2 + EPSILON)
            for dt in range(2):
                for kb in range(CH_BLOCKS):
                    for s in range(8):
                        d = dt * 8 + s
                        dvec = jnp.full((16,), d, dtype=jnp.int32)
                        for cc in range(8):
                            rvec = kb * BLK + cc * 16 + iota
                            vals = plsc.load_gather(rows_v, [rvec, dvec])
                            obuf[dt, kb * 8 + s, pl.ds(cc * 16, 16)] = vals * g
            row0 = (f * 2) * (bt_per_f * 8) + bt0 * 8
            for dt in range(2):
                pltpu.sync_copy(
                    obuf.at[dt],
                    out_hbm.at[pl.ds(row0 + dt * bt_per_f * 8, CH_BLOCKS * 8)],
                )
            return carry

        lax.fori_loop(0, n_chunks, chunk_body, 0)

    return k


def kernel(x, arch, embedding):
    batch, fields = x.shape
    idx_f = x.T.reshape(-1)
    arch16 = jnp.repeat(arch, LATENT_DIM)
    out_k = _build(batch)(idx_f, arch16, embedding)
    return (
        out_k.reshape(fields, 2, batch // 128, 8, 128)
        .transpose(2, 4, 0, 1, 3)
        .reshape(batch, fields, LATENT_DIM)
    )
